# baseline (speedup 1.0000x reference)
import functools

import jax
import jax.numpy as jnp
from jax import lax
from jax.experimental import pallas as pl
from jax.experimental.pallas import tpu as pltpu
from jax.experimental.pallas import tpu_sc as plsc

R_CH = 2


@functools.cache
def _build(BATCH: int, HIST: int, V: int, DP: int):
    info = plsc.get_sparse_core_info()
    NC, NS = info.num_cores, info.num_subcores
    NW = NC * NS
    rows_per_w = BATCH // NW
    n_chunks = rows_per_w // R_CH
    assert BATCH % NW == 0 and rows_per_w % R_CH == 0 and n_chunks % 2 == 0
    h_splits = []
    h0 = 0
    while h0 < HIST:
        h_splits.append((h0, min(128, HIST - h0)))
        h0 += 128

    mesh = plsc.VectorSubcoreMesh(core_axis_name="c", subcore_axis_name="s")

    @functools.partial(
        pl.kernel,
        out_type=jax.ShapeDtypeStruct((BATCH, HIST, DP), jnp.float32),
        mesh=mesh,
        scratch_types=[
            pltpu.VMEM((rows_per_w, HIST), jnp.int32),
            pltpu.VMEM((R_CH, HIST, DP), jnp.float32),
            pltpu.VMEM((R_CH, HIST, DP), jnp.float32),
            pltpu.SemaphoreType.DMA,
            pltpu.SemaphoreType.DMA,
            pltpu.SemaphoreType.DMA,
            pltpu.SemaphoreType.DMA,
        ],
        compiler_params=pltpu.CompilerParams(use_tc_tiling_on_sc=False),
    )
    def gather_kernel(table_hbm, x_hbm, out_hbm, idx_v, buf0, buf1,
                      gsem0, gsem1, osem0, osem1):
        wid = lax.axis_index("s") * NC + lax.axis_index("c")
        base = pl.multiple_of(wid * rows_per_w, rows_per_w)
        bufs = (buf0, buf1)
        gsems = (gsem0, gsem1)
        osems = (osem0, osem1)

        pltpu.sync_copy(x_hbm.at[pl.ds(base, rows_per_w)], idx_v)

        def fire_g(g, b):
            for j in range(R_CH):
                r = g * R_CH + j
                for (h0, hn) in h_splits:
                    pltpu.async_copy(
                        table_hbm.at[idx_v.at[r, pl.ds(h0, hn)]],
                        bufs[b].at[j, pl.ds(h0, hn)],
                        gsems[b])

        def wait_g(b):
            pltpu.make_async_copy(
                out_hbm.at[pl.ds(0, R_CH)], bufs[b], gsems[b]).wait()

        def fire_out(g, b):
            cb = pl.multiple_of(base + g * R_CH, R_CH)
            pltpu.async_copy(bufs[b], out_hbm.at[pl.ds(cb, R_CH)], osems[b])

        def wait_out(b):
            pltpu.make_async_copy(
                bufs[b], out_hbm.at[pl.ds(0, R_CH)], osems[b]).wait()

        def pair_body(i, carry):
            for b in (0, 1):
                g = 2 * i + b
                wait_out(1 - b)
                fire_g(g + 1, 1 - b)
                wait_g(b)
                fire_out(g, b)
            return carry

        fire_g(0, 0)
        fire_g(1, 1)
        wait_g(0)
        fire_out(0, 0)
        wait_out(0)
        fire_g(2, 0)
        wait_g(1)
        fire_out(1, 1)

        lax.fori_loop(1, n_chunks // 2 - 1, pair_body, 0)

        g = n_chunks - 2
        wait_out(1)
        fire_g(g + 1, 1)
        wait_g(0)
        fire_out(g, 0)
        wait_g(1)
        fire_out(g + 1, 1)
        wait_out(0)
        wait_out(1)

    return gather_kernel


def kernel(x, table):
    V, D = table.shape
    table_p = jnp.broadcast_to(table[:, None, :], (V, 2, D)).reshape(V, 2 * D)
    out_p = _build(x.shape[0], x.shape[1], V, 128)(table_p, x.astype(jnp.int32))
    return out_p[..., :D]


# speedup vs baseline: 1.2607x; 1.2607x over previous
import functools

import jax
import jax.numpy as jnp
from jax import lax
from jax.experimental import pallas as pl
from jax.experimental.pallas import tpu as pltpu
from jax.experimental.pallas import tpu_sc as plsc

R_CH = 2


@functools.cache
def _build(BATCH: int, HIST: int, V: int, DP: int):
    info = plsc.get_sparse_core_info()
    NC, NS = info.num_cores, info.num_subcores
    NW = NC * NS
    rows_per_w = BATCH // NW
    n_chunks = rows_per_w // R_CH
    assert BATCH % NW == 0 and rows_per_w % R_CH == 0 and n_chunks % 2 == 0
    h_splits = []
    h0 = 0
    while h0 < HIST:
        h_splits.append((h0, min(128, HIST - h0)))
        h0 += 128

    mesh = plsc.VectorSubcoreMesh(core_axis_name="c", subcore_axis_name="s")

    @functools.partial(
        pl.kernel,
        out_type=jax.ShapeDtypeStruct((BATCH, HIST, DP), jnp.float32),
        mesh=mesh,
        scratch_types=[
            pltpu.VMEM((rows_per_w, HIST), jnp.int32),
            pltpu.VMEM((R_CH, HIST, DP // 2), jnp.float32),
            pltpu.VMEM((R_CH, HIST, DP // 2), jnp.float32),
            pltpu.SemaphoreType.DMA,
            pltpu.SemaphoreType.DMA,
            pltpu.SemaphoreType.DMA,
            pltpu.SemaphoreType.DMA,
        ],
        compiler_params=pltpu.CompilerParams(use_tc_tiling_on_sc=False),
    )
    def gather_kernel(table_hbm, x_hbm, out_hbm, idx_v, buf0, buf1,
                      gsem0, gsem1, osem0, osem1):
        wid = lax.axis_index("s") * NC + lax.axis_index("c")
        base = pl.multiple_of(wid * rows_per_w, rows_per_w)
        bufs = (buf0, buf1)
        gsems = (gsem0, gsem1)
        osems = (osem0, osem1)

        pltpu.sync_copy(x_hbm.at[pl.ds(base, rows_per_w)], idx_v)

        def fire_g(g, b):
            for j in range(R_CH):
                r = g * R_CH + j
                for (h0, hn) in h_splits:
                    pltpu.async_copy(
                        table_hbm.at[idx_v.at[r, pl.ds(h0, hn)]],
                        bufs[b].at[j, pl.ds(h0, hn)],
                        gsems[b])

        def wait_g(b):
            pltpu.make_async_copy(
                out_hbm.at[pl.ds(0, R_CH), :, pl.ds(0, DP // 2)],
                bufs[b], gsems[b]).wait()

        def fire_out(g, b):
            cb = pl.multiple_of(base + g * R_CH, R_CH)
            pltpu.async_copy(
                bufs[b],
                out_hbm.at[pl.ds(cb, R_CH), :, pl.ds(0, DP // 2)],
                osems[b])

        def wait_out(b):
            pltpu.make_async_copy(
                bufs[b], out_hbm.at[pl.ds(0, R_CH), :, pl.ds(0, DP // 2)],
                osems[b]).wait()

        def pair_body(i, carry):
            for b in (0, 1):
                g = 2 * i + b
                wait_out(1 - b)
                fire_g(g + 1, 1 - b)
                wait_g(b)
                fire_out(g, b)
            return carry

        fire_g(0, 0)
        fire_g(1, 1)
        wait_g(0)
        fire_out(0, 0)
        wait_out(0)
        fire_g(2, 0)
        wait_g(1)
        fire_out(1, 1)

        lax.fori_loop(1, n_chunks // 2 - 1, pair_body, 0)

        g = n_chunks - 2
        wait_out(1)
        fire_g(g + 1, 1)
        wait_g(0)
        fire_out(g, 0)
        wait_g(1)
        fire_out(g + 1, 1)
        wait_out(0)
        wait_out(1)

    return gather_kernel


def kernel(x, table):
    V, D = table.shape
    table_p = jnp.pad(table, ((0, 0), (0, 128 - D))).reshape(2 * V, D)
    x2 = x.astype(jnp.int32) * 2
    out_p = _build(x.shape[0], x.shape[1], V, 128)(table_p, x2)
    return out_p[..., :D]


# speedup vs baseline: 1.2632x; 1.0020x over previous
import functools

import jax
import jax.numpy as jnp
from jax import lax
from jax.experimental import pallas as pl
from jax.experimental.pallas import tpu as pltpu
from jax.experimental.pallas import tpu_sc as plsc

R_CH = 4


@functools.cache
def _build(BATCH: int, HIST: int, V: int, DP: int):
    info = plsc.get_sparse_core_info()
    NC, NS = info.num_cores, info.num_subcores
    NW = NC * NS
    rows_per_w = BATCH // NW
    n_chunks = rows_per_w // R_CH
    assert BATCH % NW == 0 and rows_per_w % R_CH == 0 and n_chunks % 2 == 0
    h_splits = []
    h0 = 0
    while h0 < HIST:
        h_splits.append((h0, min(128, HIST - h0)))
        h0 += 128

    mesh = plsc.VectorSubcoreMesh(core_axis_name="c", subcore_axis_name="s")

    @functools.partial(
        pl.kernel,
        out_type=jax.ShapeDtypeStruct((BATCH, HIST, DP), jnp.float32),
        mesh=mesh,
        scratch_types=[
            pltpu.VMEM((rows_per_w, HIST), jnp.int32),
            pltpu.VMEM((R_CH, HIST, DP // 2), jnp.float32),
            pltpu.VMEM((R_CH, HIST, DP // 2), jnp.float32),
            pltpu.SemaphoreType.DMA,
            pltpu.SemaphoreType.DMA,
            pltpu.SemaphoreType.DMA,
            pltpu.SemaphoreType.DMA,
        ],
        compiler_params=pltpu.CompilerParams(use_tc_tiling_on_sc=False),
    )
    def gather_kernel(table_hbm, x_hbm, out_hbm, idx_v, buf0, buf1,
                      gsem0, gsem1, osem0, osem1):
        wid = lax.axis_index("s") * NC + lax.axis_index("c")
        base = pl.multiple_of(wid * rows_per_w, rows_per_w)
        bufs = (buf0, buf1)
        gsems = (gsem0, gsem1)
        osems = (osem0, osem1)

        pltpu.sync_copy(x_hbm.at[pl.ds(base, rows_per_w)], idx_v)

        def fire_g(g, b):
            for j in range(R_CH):
                r = g * R_CH + j
                for (h0, hn) in h_splits:
                    pltpu.async_copy(
                        table_hbm.at[idx_v.at[r, pl.ds(h0, hn)]],
                        bufs[b].at[j, pl.ds(h0, hn)],
                        gsems[b])

        def wait_g(b):
            pltpu.make_async_copy(
                out_hbm.at[pl.ds(0, R_CH), :, pl.ds(0, DP // 2)],
                bufs[b], gsems[b]).wait()

        def fire_out(g, b):
            cb = pl.multiple_of(base + g * R_CH, R_CH)
            pltpu.async_copy(
                bufs[b],
                out_hbm.at[pl.ds(cb, R_CH), :, pl.ds(0, DP // 2)],
                osems[b])

        def wait_out(b):
            pltpu.make_async_copy(
                bufs[b], out_hbm.at[pl.ds(0, R_CH), :, pl.ds(0, DP // 2)],
                osems[b]).wait()

        def pair_body(i, carry):
            for b in (0, 1):
                g = 2 * i + b
                wait_out(1 - b)
                fire_g(g + 1, 1 - b)
                wait_g(b)
                fire_out(g, b)
            return carry

        fire_g(0, 0)
        fire_g(1, 1)
        wait_g(0)
        fire_out(0, 0)
        wait_out(0)
        fire_g(2, 0)
        wait_g(1)
        fire_out(1, 1)

        lax.fori_loop(1, n_chunks // 2 - 1, pair_body, 0)

        g = n_chunks - 2
        wait_out(1)
        fire_g(g + 1, 1)
        wait_g(0)
        fire_out(g, 0)
        wait_g(1)
        fire_out(g + 1, 1)
        wait_out(0)
        wait_out(1)

    return gather_kernel


def kernel(x, table):
    V, D = table.shape
    table_p = jnp.pad(table, ((0, 0), (0, 128 - D))).reshape(2 * V, D)
    x2 = x.astype(jnp.int32) * 2
    out_p = _build(x.shape[0], x.shape[1], V, 128)(table_p, x2)
    return out_p[..., :D]
